# baseline (speedup 1.0000x reference)
import functools

import jax
import jax.numpy as jnp
from jax import lax
from jax.experimental import pallas as pl
from jax.experimental.pallas import tpu as pltpu
from jax.experimental.pallas import tpu_sc as plsc

_VOCAB = 1000000
_DIM = 32
_CTX = 200
_GA = 128
_GB = _CTX - _GA
_VB = 8192
_NB = -(-_VOCAB // _VB)
_NS = 128


def _sc_gather_mean(idx_hbm, table_hbm, out_hbm, idx_a, idx_b, rows_a,
                    rows_b, acc_v, sem):
    wid = lax.axis_index("s") * 2 + lax.axis_index("c")

    @pl.when(wid == 0)
    def _():
        pltpu.sync_copy(idx_hbm.at[pl.ds(0, _GA)], idx_a)
        pltpu.sync_copy(idx_hbm.at[pl.ds(_GA, _GB)], idx_b)
        cp_a = pltpu.make_async_copy(table_hbm.at[idx_a], rows_a, sem)
        cp_b = pltpu.make_async_copy(table_hbm.at[idx_b], rows_b, sem)
        cp_a.start()
        cp_b.start()
        cp_a.wait()
        cp_b.wait()
        for d in range(_DIM // 16):
            acc = rows_a[0, pl.ds(d * 16, 16)]
            for r in range(1, _GA):
                acc = acc + rows_a[r, pl.ds(d * 16, 16)]
            for r in range(_GB):
                acc = acc + rows_b[r, pl.ds(d * 16, 16)]
            acc_v[pl.ds(d * 16, 16)] = acc * (1.0 / _CTX)
        pltpu.sync_copy(acc_v, out_hbm)


def _gather_mean(inputs, emb_table):
    mesh = plsc.VectorSubcoreMesh(core_axis_name="c", subcore_axis_name="s")
    k = functools.partial(
        pl.kernel,
        mesh=mesh,
        compiler_params=pltpu.CompilerParams(use_tc_tiling_on_sc=False),
        out_type=jax.ShapeDtypeStruct((_DIM,), jnp.float32),
        scratch_types=[
            pltpu.VMEM((_GA,), jnp.int32),
            pltpu.VMEM((_GB,), jnp.int32),
            pltpu.VMEM((_GA, _DIM), jnp.float32),
            pltpu.VMEM((_GB, _DIM), jnp.float32),
            pltpu.VMEM((_DIM,), jnp.float32),
            pltpu.SemaphoreType.DMA,
        ],
    )(_sc_gather_mean)
    return k(inputs, emb_table)


def _proj_kernel(x_ref, w_ref, b_ref, o_ref, y_s, m_s, z_s):
    p = pl.program_id(0)
    j = pl.program_id(1)

    @pl.when(p == 0)
    def _():
        @pl.when(j == 0)
        def _():
            m_s[...] = jnp.full_like(m_s, -jnp.inf)
            y_s[pl.ds(_NS - 8, 8), :] = jnp.full((8, _VB), -jnp.inf,
                                                 jnp.float32)

        y = jax.lax.dot_general(
            x_ref[...], w_ref[...],
            (((1,), (1,)), ((), ())),
            preferred_element_type=jnp.float32,
        ) + b_ref[...][None]
        col = j * _VB + jax.lax.broadcasted_iota(jnp.int32, (1, _VB), 1)
        y = jnp.where(col < _VOCAB, y, -jnp.inf)
        y_s[pl.ds(j, 1), :] = y
        m_s[...] = jnp.maximum(m_s[...], y)

        @pl.when(j == _NB - 1)
        def _():
            m = jnp.max(m_s[...])

            def body(t, s8):
                return s8 + jnp.exp(y_s[pl.ds(8 * t, 8), :] - m)

            s8 = lax.fori_loop(
                0, _NS // 8, body, jnp.zeros((8, _VB), jnp.float32))
            z_s[0, 0] = m + jnp.log(jnp.sum(s8))

    @pl.when(p == 1)
    def _():
        o_ref[...] = y_s[pl.ds(j, 1), :] - z_s[0, 0]


@jax.jit
def kernel(inputs, emb_table, lin_w, lin_b):
    x = _gather_mean(inputs, emb_table).reshape(1, _DIM)

    out = pl.pallas_call(
        _proj_kernel,
        grid=(2, _NB),
        in_specs=[
            pl.BlockSpec((1, _DIM), lambda p, j: (0, 0)),
            pl.BlockSpec((_VB, _DIM), lambda p, j: (j * (1 - p), 0)),
            pl.BlockSpec((_VB,), lambda p, j: (j * (1 - p),)),
        ],
        out_specs=pl.BlockSpec((1, _VB), lambda p, j: (0, j * p)),
        out_shape=jax.ShapeDtypeStruct((1, _VOCAB), jnp.float32),
        scratch_shapes=[
            pltpu.VMEM((_NS, _VB), jnp.float32),
            pltpu.VMEM((1, _VB), jnp.float32),
            pltpu.SMEM((1, 1), jnp.float32),
        ],
    )(x, lin_w, lin_b)

    return out


# speedup vs baseline: 1.2624x; 1.2624x over previous
import functools

import jax
import jax.numpy as jnp
from jax import lax
from jax.experimental import pallas as pl
from jax.experimental.pallas import tpu as pltpu
from jax.experimental.pallas import tpu_sc as plsc

_VOCAB = 1000000
_DIM = 32
_CTX = 200
_VB = 8192
_NB = -(-_VOCAB // _VB)
_NS = 128


def _sc_gather_mean(idx_hbm, table_hbm, out_hbm, idx_v, rows_v,
                    acc_v, sem):
    wid = lax.axis_index("s") * 2 + lax.axis_index("c")

    @pl.when(wid == 0)
    def _():
        pltpu.sync_copy(idx_hbm, idx_v.at[pl.ds(0, _CTX)])
        for c in range(0, _CTX, 16):
            vec = idx_v[pl.ds(c, 16)]
            for i in range(min(16, _CTX - c)):
                pltpu.make_async_copy(
                    table_hbm.at[pl.ds(vec[i], 1), :],
                    rows_v.at[pl.ds(c + i, 1), :], sem).start()
        pltpu.make_async_copy(
            table_hbm.at[pl.ds(0, _CTX), :], rows_v, sem).wait()
        for d in range(_DIM // 16):
            acc = rows_v[0, pl.ds(d * 16, 16)]
            for r in range(1, _CTX):
                acc = acc + rows_v[r, pl.ds(d * 16, 16)]
            acc_v[pl.ds(d * 16, 16)] = acc * (1.0 / _CTX)
        pltpu.sync_copy(acc_v, out_hbm)


def _gather_mean(inputs, emb_table):
    mesh = plsc.VectorSubcoreMesh(core_axis_name="c", subcore_axis_name="s")
    k = functools.partial(
        pl.kernel,
        mesh=mesh,
        out_type=jax.ShapeDtypeStruct((_DIM,), jnp.float32),
        scratch_types=[
            pltpu.VMEM((208,), jnp.int32),
            pltpu.VMEM((_CTX, _DIM), jnp.float32),
            pltpu.VMEM((_DIM,), jnp.float32),
            pltpu.SemaphoreType.DMA,
        ],
    )(_sc_gather_mean)
    return k(inputs, emb_table)


def _proj_kernel(x_ref, w_ref, b_ref, o_ref, y_s, m_s, z_s):
    p = pl.program_id(0)
    j = pl.program_id(1)

    @pl.when(p == 0)
    def _():
        @pl.when(j == 0)
        def _():
            m_s[...] = jnp.full_like(m_s, -jnp.inf)
            y_s[pl.ds(_NS - 8, 8), :] = jnp.full((8, _VB), -jnp.inf,
                                                 jnp.float32)

        y = jax.lax.dot_general(
            x_ref[...], w_ref[...],
            (((1,), (1,)), ((), ())),
            preferred_element_type=jnp.float32,
        ) + b_ref[...][None]
        col = j * _VB + jax.lax.broadcasted_iota(jnp.int32, (1, _VB), 1)
        y = jnp.where(col < _VOCAB, y, -jnp.inf)
        y_s[pl.ds(j, 1), :] = y
        m_s[...] = jnp.maximum(m_s[...], y)

        @pl.when(j == _NB - 1)
        def _():
            m = jnp.max(m_s[...])

            def body(t, s8):
                return s8 + jnp.exp(y_s[pl.ds(8 * t, 8), :] - m)

            s8 = lax.fori_loop(
                0, _NS // 8, body, jnp.zeros((8, _VB), jnp.float32))
            z_s[0, 0] = m + jnp.log(jnp.sum(s8))

    @pl.when(p == 1)
    def _():
        o_ref[...] = y_s[pl.ds(j, 1), :] - z_s[0, 0]


@jax.jit
def kernel(inputs, emb_table, lin_w, lin_b):
    x = _gather_mean(inputs, emb_table).reshape(1, _DIM)

    out = pl.pallas_call(
        _proj_kernel,
        grid=(2, _NB),
        in_specs=[
            pl.BlockSpec((1, _DIM), lambda p, j: (0, 0)),
            pl.BlockSpec((_VB, _DIM), lambda p, j: (j * (1 - p), 0)),
            pl.BlockSpec((_VB,), lambda p, j: (j * (1 - p),)),
        ],
        out_specs=pl.BlockSpec((1, _VB), lambda p, j: (0, j * p)),
        out_shape=jax.ShapeDtypeStruct((1, _VOCAB), jnp.float32),
        scratch_shapes=[
            pltpu.VMEM((_NS, _VB), jnp.float32),
            pltpu.VMEM((1, _VB), jnp.float32),
            pltpu.SMEM((1, 1), jnp.float32),
        ],
    )(x, lin_w, lin_b)

    return out


# speedup vs baseline: 1.3581x; 1.0758x over previous
import functools

import jax
import jax.numpy as jnp
from jax import lax
from jax.experimental import pallas as pl
from jax.experimental.pallas import tpu as pltpu
from jax.experimental.pallas import tpu_sc as plsc

_VOCAB = 1000000
_DIM = 32
_CTX = 200
_VB = 32768
_NB = -(-_VOCAB // _VB)
_NS = 32


def _sc_gather_mean(idx_hbm, table_hbm, out_hbm, idx_v, rows_v,
                    acc_v, sem):
    wid = lax.axis_index("s") * 2 + lax.axis_index("c")

    @pl.when(wid == 0)
    def _():
        pltpu.sync_copy(idx_hbm, idx_v.at[pl.ds(0, _CTX)])
        for c in range(0, _CTX, 16):
            vec = idx_v[pl.ds(c, 16)]
            for i in range(min(16, _CTX - c)):
                pltpu.make_async_copy(
                    table_hbm.at[pl.ds(vec[i], 1), :],
                    rows_v.at[pl.ds(c + i, 1), :], sem).start()
        pltpu.make_async_copy(
            table_hbm.at[pl.ds(0, _CTX), :], rows_v, sem).wait()
        for d in range(_DIM // 16):
            acc = rows_v[0, pl.ds(d * 16, 16)]
            for r in range(1, _CTX):
                acc = acc + rows_v[r, pl.ds(d * 16, 16)]
            acc_v[pl.ds(d * 16, 16)] = acc * (1.0 / _CTX)
        pltpu.sync_copy(acc_v, out_hbm)


def _gather_mean(inputs, emb_table):
    mesh = plsc.VectorSubcoreMesh(core_axis_name="c", subcore_axis_name="s")
    k = functools.partial(
        pl.kernel,
        mesh=mesh,
        out_type=jax.ShapeDtypeStruct((_DIM,), jnp.float32),
        scratch_types=[
            pltpu.VMEM((208,), jnp.int32),
            pltpu.VMEM((_CTX, _DIM), jnp.float32),
            pltpu.VMEM((_DIM,), jnp.float32),
            pltpu.SemaphoreType.DMA,
        ],
    )(_sc_gather_mean)
    return k(inputs, emb_table)


def _proj_kernel(x_ref, w_ref, b_ref, o_ref, y_s, m_s, z_s):
    p = pl.program_id(0)
    j = pl.program_id(1)

    @pl.when(p == 0)
    def _():
        @pl.when(j == 0)
        def _():
            m_s[...] = jnp.full_like(m_s, -jnp.inf)
            y_s[pl.ds(_NS - 8, 8), :] = jnp.full((8, _VB), -jnp.inf,
                                                 jnp.float32)

        y = jax.lax.dot_general(
            x_ref[...], w_ref[...],
            (((1,), (1,)), ((), ())),
            preferred_element_type=jnp.float32,
        ) + b_ref[...][None]
        col = j * _VB + jax.lax.broadcasted_iota(jnp.int32, (1, _VB), 1)
        y = jnp.where(col < _VOCAB, y, -jnp.inf)
        y_s[pl.ds(j, 1), :] = y
        m_s[...] = jnp.maximum(m_s[...], y)

        @pl.when(j == _NB - 1)
        def _():
            m = jnp.max(m_s[...])

            def body(t, s8):
                return s8 + jnp.exp(y_s[pl.ds(8 * t, 8), :] - m)

            s8 = lax.fori_loop(
                0, _NS // 8, body, jnp.zeros((8, _VB), jnp.float32))
            z_s[0, 0] = m + jnp.log(jnp.sum(s8))

    @pl.when(p == 1)
    def _():
        o_ref[...] = y_s[pl.ds(j, 1), :] - z_s[0, 0]


@jax.jit
def kernel(inputs, emb_table, lin_w, lin_b):
    x = _gather_mean(inputs, emb_table).reshape(1, _DIM)

    out = pl.pallas_call(
        _proj_kernel,
        grid=(2, _NB),
        in_specs=[
            pl.BlockSpec((1, _DIM), lambda p, j: (0, 0)),
            pl.BlockSpec((_VB, _DIM), lambda p, j: (j * (1 - p), 0)),
            pl.BlockSpec((_VB,), lambda p, j: (j * (1 - p),)),
        ],
        out_specs=pl.BlockSpec((1, _VB), lambda p, j: (0, j * p)),
        out_shape=jax.ShapeDtypeStruct((1, _VOCAB), jnp.float32),
        scratch_shapes=[
            pltpu.VMEM((_NS, _VB), jnp.float32),
            pltpu.VMEM((1, _VB), jnp.float32),
            pltpu.SMEM((1, 1), jnp.float32),
        ],
    )(x, lin_w, lin_b)

    return out
